# initial kernel scaffold (unmeasured)
import jax
import jax.numpy as jnp
from jax import lax
from jax.experimental import pallas as pl
from jax.experimental.pallas import tpu as pltpu

B, S, H, Dh, Dr = 2, 512, 16, 128, 32
D = 2048
DC_SH = 128
F32 = jnp.float32


def _dot(a, b):
    return jnp.dot(a, b, preferred_element_type=F32)


def _kv_comm_body(x_ref, wdkv_ref, wuk_ref, wuv_ref, wkr_ref,
                  k_ref, v_ref, kr_ref,
                  c_ref, peer_c, peer_wuk, peer_wuv,
                  send_sems, recv_sems):
    my_x = lax.axis_index("x")
    my_y = lax.axis_index("y")
    my_z = lax.axis_index("z")
    peer = (my_x, 1 - my_y, my_z)

    barrier_sem = pltpu.get_barrier_semaphore()
    pl.semaphore_signal(barrier_sem, inc=1, device_id=peer,
                        device_id_type=pl.DeviceIdType.MESH)
    pl.semaphore_wait(barrier_sem, 1)

    c_ref[...] = _dot(x_ref[...], wdkv_ref[...])

    rdmas = []
    for i, (src, dst) in enumerate(((c_ref, peer_c),
                                    (wuk_ref, peer_wuk),
                                    (wuv_ref, peer_wuv))):
        rdma = pltpu.make_async_remote_copy(
            src_ref=src, dst_ref=dst,
            send_sem=send_sems.at[i], recv_sem=recv_sems.at[i],
            device_id=peer, device_id_type=pl.DeviceIdType.MESH)
        rdma.start()
        rdmas.append(rdma)

    kr_ref[...] = _dot(x_ref[...], wkr_ref[...])
    k_ref[...] = _dot(c_ref[...], wuk_ref[...])
    v_ref[...] = _dot(c_ref[...], wuv_ref[...])

    for rdma in rdmas:
        rdma.wait()

    k_ref[...] += _dot(peer_c[...], peer_wuk[...])
    v_ref[...] += _dot(peer_c[...], peer_wuv[...])


def _attn_body(x_ref, wq_ref, wqr_ref, k_ref, v_ref, kr_ref, o_ref):
    xb = x_ref[0]
    q = _dot(xb, wq_ref[...])
    qr = _dot(xb, wqr_ref[...])
    k = k_ref[0]
    v = v_ref[0]
    kr = kr_ref[0]
    scale = (Dh + Dr) ** -0.5
    contract_last = (((1,), (1,)), ((), ()))
    scores = (lax.dot_general(q, k, contract_last, preferred_element_type=F32)
              + lax.dot_general(qr, kr, contract_last,
                                preferred_element_type=F32)) * scale
    m = jnp.max(scores, axis=-1, keepdims=True)
    p = jnp.exp(scores - m)
    p = p / jnp.sum(p, axis=-1, keepdims=True)
    o_ref[0] = _dot(p, v)


def _out_body(o_ref, wo_ref, out_ref):
    out_ref[...] = _dot(o_ref[...], wo_ref[...])


def kernel(x, Wdkv, Wuk, Wuv, Wq, Wqr, Wkr, Wo):
    x2 = x.reshape(B * S, D)

    k, v, kr = pl.pallas_call(
        _kv_comm_body,
        out_shape=(
            jax.ShapeDtypeStruct((B * S, H * Dh), F32),
            jax.ShapeDtypeStruct((B * S, H * Dh), F32),
            jax.ShapeDtypeStruct((B * S, Dr), F32),
        ),
        in_specs=[pl.BlockSpec(memory_space=pltpu.VMEM)] * 5,
        out_specs=[pl.BlockSpec(memory_space=pltpu.VMEM)] * 3,
        scratch_shapes=[
            pltpu.VMEM((B * S, DC_SH), F32),
            pltpu.VMEM((B * S, DC_SH), F32),
            pltpu.VMEM((DC_SH, H * Dh), F32),
            pltpu.VMEM((DC_SH, H * Dh), F32),
            pltpu.SemaphoreType.DMA((3,)),
            pltpu.SemaphoreType.DMA((3,)),
        ],
        compiler_params=pltpu.CompilerParams(collective_id=0),
    )(x2, Wdkv, Wuk, Wuv, Wkr)

    k3 = k.reshape(B, S, H * Dh)
    v3 = v.reshape(B, S, H * Dh)
    kr3 = kr.reshape(B, S, Dr)

    o = pl.pallas_call(
        _attn_body,
        grid=(B, H),
        out_shape=jax.ShapeDtypeStruct((B, S, H * Dh), F32),
        in_specs=[
            pl.BlockSpec((1, S, D), lambda b, h: (b, 0, 0)),
            pl.BlockSpec((D, Dh), lambda b, h: (0, h)),
            pl.BlockSpec((D, Dr), lambda b, h: (0, h)),
            pl.BlockSpec((1, S, Dh), lambda b, h: (b, 0, h)),
            pl.BlockSpec((1, S, Dh), lambda b, h: (b, 0, h)),
            pl.BlockSpec((1, S, Dr), lambda b, h: (b, 0, 0)),
        ],
        out_specs=pl.BlockSpec((1, S, Dh), lambda b, h: (b, 0, h)),
    )(x, Wq, Wqr, k3, v3, kr3)

    o2 = o.reshape(B * S, H * Dh)
    n_wo_blocks = 4
    wo_blk = D // n_wo_blocks
    out = pl.pallas_call(
        _out_body,
        grid=(n_wo_blocks,),
        out_shape=jax.ShapeDtypeStruct((B * S, D), F32),
        in_specs=[
            pl.BlockSpec((B * S, H * Dh), lambda j: (0, 0)),
            pl.BlockSpec((H * Dh, wo_blk), lambda j: (0, j)),
        ],
        out_specs=pl.BlockSpec((B * S, wo_blk), lambda j: (0, j)),
    )(o2, Wo)

    return out.reshape(B, S, D)


# baseline (device time: 150988 ns/iter reference)
import jax
import jax.numpy as jnp
from jax import lax
from jax.experimental import pallas as pl
from jax.experimental.pallas import tpu as pltpu

B, S, H, Dh, Dr = 2, 512, 16, 128, 32
D = 2048
DC_SH = 128
F32 = jnp.float32


def _dot(a, b):
    return jnp.dot(a, b, preferred_element_type=F32)


def _kv_comm_body(x_ref, wdkv_ref, wuk_ref, wuv_ref, wkr_ref,
                  k_ref, v_ref, kr_ref,
                  c_ref, peer_c, peer_wuk, peer_wuv,
                  send_sems, recv_sems):
    my_x = lax.axis_index("x")
    my_y = lax.axis_index("y")
    my_z = lax.axis_index("z")
    peer = (my_x, 1 - my_y, my_z)

    barrier_sem = pltpu.get_barrier_semaphore()
    pl.semaphore_signal(barrier_sem, inc=1, device_id=peer,
                        device_id_type=pl.DeviceIdType.MESH)
    pl.semaphore_wait(barrier_sem, 1)

    c_ref[...] = _dot(x_ref[...], wdkv_ref[...])

    rdmas = []
    for i, (src, dst) in enumerate(((c_ref, peer_c),
                                    (wuk_ref, peer_wuk),
                                    (wuv_ref, peer_wuv))):
        rdma = pltpu.make_async_remote_copy(
            src_ref=src, dst_ref=dst,
            send_sem=send_sems.at[i], recv_sem=recv_sems.at[i],
            device_id=peer, device_id_type=pl.DeviceIdType.MESH)
        rdma.start()
        rdmas.append(rdma)

    kr_ref[...] = _dot(x_ref[...], wkr_ref[...])
    k_ref[...] = _dot(c_ref[...], wuk_ref[...])
    v_ref[...] = _dot(c_ref[...], wuv_ref[...])

    for rdma in rdmas:
        rdma.wait()

    k_ref[...] += _dot(peer_c[...], peer_wuk[...])
    v_ref[...] += _dot(peer_c[...], peer_wuv[...])


def _attn_body(x_ref, wq_ref, wqr_ref, k_ref, v_ref, kr_ref, o_ref):
    xb = x_ref[0]
    q = _dot(xb, wq_ref[...])
    qr = _dot(xb, wqr_ref[0])
    k = k_ref[0]
    v = v_ref[0]
    kr = kr_ref[0]
    scale = (Dh + Dr) ** -0.5
    contract_last = (((1,), (1,)), ((), ()))
    scores = (lax.dot_general(q, k, contract_last, preferred_element_type=F32)
              + lax.dot_general(qr, kr, contract_last,
                                preferred_element_type=F32)) * scale
    m = jnp.max(scores, axis=-1, keepdims=True)
    p = jnp.exp(scores - m)
    p = p / jnp.sum(p, axis=-1, keepdims=True)
    o_ref[0] = _dot(p, v)


def _out_body(o_ref, wo_ref, out_ref):
    out_ref[...] = _dot(o_ref[...], wo_ref[...])


def kernel(x, Wdkv, Wuk, Wuv, Wq, Wqr, Wkr, Wo):
    x2 = x.reshape(B * S, D)

    k, v, kr = pl.pallas_call(
        _kv_comm_body,
        out_shape=(
            jax.ShapeDtypeStruct((B * S, H * Dh), F32),
            jax.ShapeDtypeStruct((B * S, H * Dh), F32),
            jax.ShapeDtypeStruct((B * S, Dr), F32),
        ),
        in_specs=[pl.BlockSpec(memory_space=pltpu.VMEM)] * 5,
        out_specs=[pl.BlockSpec(memory_space=pltpu.VMEM)] * 3,
        scratch_shapes=[
            pltpu.VMEM((B * S, DC_SH), F32),
            pltpu.VMEM((B * S, DC_SH), F32),
            pltpu.VMEM((DC_SH, H * Dh), F32),
            pltpu.VMEM((DC_SH, H * Dh), F32),
            pltpu.SemaphoreType.DMA((3,)),
            pltpu.SemaphoreType.DMA((3,)),
        ],
        compiler_params=pltpu.CompilerParams(collective_id=0),
    )(x2, Wdkv, Wuk, Wuv, Wkr)

    k3 = k.reshape(B, S, H * Dh)
    v3 = v.reshape(B, S, H * Dh)
    kr3 = kr.reshape(B, S, Dr)
    wqr_t = Wqr.reshape(D, H, Dr).transpose(1, 0, 2)

    o = pl.pallas_call(
        _attn_body,
        grid=(B, H),
        out_shape=jax.ShapeDtypeStruct((B, S, H * Dh), F32),
        in_specs=[
            pl.BlockSpec((1, S, D), lambda b, h: (b, 0, 0)),
            pl.BlockSpec((D, Dh), lambda b, h: (0, h)),
            pl.BlockSpec((1, D, Dr), lambda b, h: (h, 0, 0)),
            pl.BlockSpec((1, S, Dh), lambda b, h: (b, 0, h)),
            pl.BlockSpec((1, S, Dh), lambda b, h: (b, 0, h)),
            pl.BlockSpec((1, S, Dr), lambda b, h: (b, 0, 0)),
        ],
        out_specs=pl.BlockSpec((1, S, Dh), lambda b, h: (b, 0, h)),
    )(x, Wq, wqr_t, k3, v3, kr3)

    o2 = o.reshape(B * S, H * Dh)
    n_wo_blocks = 4
    wo_blk = D // n_wo_blocks
    out = pl.pallas_call(
        _out_body,
        grid=(n_wo_blocks,),
        out_shape=jax.ShapeDtypeStruct((B * S, D), F32),
        in_specs=[
            pl.BlockSpec((B * S, H * Dh), lambda j: (0, 0)),
            pl.BlockSpec((H * Dh, wo_blk), lambda j: (0, j)),
        ],
        out_specs=pl.BlockSpec((B * S, wo_blk), lambda j: (0, j)),
    )(o2, Wo)

    return out.reshape(B, S, D)


# device time: 130457 ns/iter; 1.1574x vs baseline; 1.1574x over previous
import jax
import jax.numpy as jnp
from jax import lax
from jax.experimental import pallas as pl
from jax.experimental.pallas import tpu as pltpu

B, S, H, Dh, Dr = 2, 512, 16, 128, 32
D = 2048
DC_SH = 128
F32 = jnp.float32
BF16 = jnp.bfloat16


def _dot(a, b):
    return jnp.dot(a, b, preferred_element_type=F32)


def _kv_comm_body(x_ref, wdkv_ref, wuk_ref, wuv_ref, wkr_ref,
                  k_ref, v_ref, kr_ref, xbf_ref,
                  c_ref, wuk_bf, wuv_bf, peer_c, peer_wuk, peer_wuv,
                  send_sems, recv_sems):
    my_x = lax.axis_index("x")
    my_y = lax.axis_index("y")
    my_z = lax.axis_index("z")
    peer = (my_x, 1 - my_y, my_z)

    barrier_sem = pltpu.get_barrier_semaphore()
    pl.semaphore_signal(barrier_sem, inc=1, device_id=peer,
                        device_id_type=pl.DeviceIdType.MESH)
    pl.semaphore_wait(barrier_sem, 1)

    x_bf = x_ref[...].astype(BF16)
    xbf_ref[...] = x_bf
    wuk_bf[...] = wuk_ref[...].astype(BF16)
    wuv_bf[...] = wuv_ref[...].astype(BF16)
    c_ref[...] = _dot(x_bf, wdkv_ref[...].astype(BF16)).astype(BF16)

    rdmas = []
    for i, (src, dst) in enumerate(((c_ref, peer_c),
                                    (wuk_bf, peer_wuk),
                                    (wuv_bf, peer_wuv))):
        rdma = pltpu.make_async_remote_copy(
            src_ref=src, dst_ref=dst,
            send_sem=send_sems.at[i], recv_sem=recv_sems.at[i],
            device_id=peer, device_id_type=pl.DeviceIdType.MESH)
        rdma.start()
        rdmas.append(rdma)

    kr_ref[...] = _dot(x_bf, wkr_ref[...].astype(BF16)).astype(BF16)
    k_local = _dot(c_ref[...], wuk_bf[...])
    v_local = _dot(c_ref[...], wuv_bf[...])

    for rdma in rdmas:
        rdma.wait()

    k_ref[...] = (k_local + _dot(peer_c[...], peer_wuk[...])).astype(BF16)
    v_ref[...] = (v_local + _dot(peer_c[...], peer_wuv[...])).astype(BF16)


def _attn_body(x_ref, wq_ref, wqr_ref, k_ref, v_ref, kr_ref, o_ref):
    xb = x_ref[0]
    scale = (Dh + Dr) ** -0.5
    q = (_dot(xb, wq_ref[...].astype(BF16)) * scale).astype(BF16)
    qr = (_dot(xb, wqr_ref[0].astype(BF16)) * scale).astype(BF16)
    contract_last = (((1,), (1,)), ((), ()))
    scores = (lax.dot_general(q, k_ref[0], contract_last,
                              preferred_element_type=F32)
              + lax.dot_general(qr, kr_ref[0], contract_last,
                                preferred_element_type=F32))
    p = jnp.exp(scores)
    recip = 1.0 / jnp.sum(p, axis=-1, keepdims=True)
    o = _dot(p.astype(BF16), v_ref[0])
    o_ref[0] = (o * recip).astype(BF16)


def _out_body(o_ref, wo_ref, out_ref):
    out_ref[...] = _dot(o_ref[...], wo_ref[...].astype(BF16))


def kernel(x, Wdkv, Wuk, Wuv, Wq, Wqr, Wkr, Wo):
    x2 = x.reshape(B * S, D)

    k, v, kr, x_bf = pl.pallas_call(
        _kv_comm_body,
        out_shape=(
            jax.ShapeDtypeStruct((B * S, H * Dh), BF16),
            jax.ShapeDtypeStruct((B * S, H * Dh), BF16),
            jax.ShapeDtypeStruct((B * S, Dr), BF16),
            jax.ShapeDtypeStruct((B * S, D), BF16),
        ),
        in_specs=[pl.BlockSpec(memory_space=pltpu.VMEM)] * 5,
        out_specs=[pl.BlockSpec(memory_space=pltpu.VMEM)] * 4,
        scratch_shapes=[
            pltpu.VMEM((B * S, DC_SH), BF16),
            pltpu.VMEM((DC_SH, H * Dh), BF16),
            pltpu.VMEM((DC_SH, H * Dh), BF16),
            pltpu.VMEM((B * S, DC_SH), BF16),
            pltpu.VMEM((DC_SH, H * Dh), BF16),
            pltpu.VMEM((DC_SH, H * Dh), BF16),
            pltpu.SemaphoreType.DMA((3,)),
            pltpu.SemaphoreType.DMA((3,)),
        ],
        compiler_params=pltpu.CompilerParams(collective_id=0),
    )(x2, Wdkv, Wuk, Wuv, Wkr)

    k3 = k.reshape(B, S, H * Dh)
    v3 = v.reshape(B, S, H * Dh)
    kr3 = kr.reshape(B, S, Dr)
    x3 = x_bf.reshape(B, S, D)
    wqr_t = Wqr.reshape(D, H, Dr).transpose(1, 0, 2)

    o = pl.pallas_call(
        _attn_body,
        grid=(B, H),
        out_shape=jax.ShapeDtypeStruct((B, S, H * Dh), BF16),
        in_specs=[
            pl.BlockSpec((1, S, D), lambda b, h: (b, 0, 0)),
            pl.BlockSpec((D, Dh), lambda b, h: (0, h)),
            pl.BlockSpec((1, D, Dr), lambda b, h: (h, 0, 0)),
            pl.BlockSpec((1, S, Dh), lambda b, h: (b, 0, h)),
            pl.BlockSpec((1, S, Dh), lambda b, h: (b, 0, h)),
            pl.BlockSpec((1, S, Dr), lambda b, h: (b, 0, 0)),
        ],
        out_specs=pl.BlockSpec((1, S, Dh), lambda b, h: (b, 0, h)),
    )(x3, Wq, wqr_t, k3, v3, kr3)

    o2 = o.reshape(B * S, H * Dh)
    n_wo_blocks = 4
    wo_blk = D // n_wo_blocks
    out = pl.pallas_call(
        _out_body,
        grid=(n_wo_blocks,),
        out_shape=jax.ShapeDtypeStruct((B * S, D), F32),
        in_specs=[
            pl.BlockSpec((B * S, H * Dh), lambda j: (0, 0)),
            pl.BlockSpec((H * Dh, wo_blk), lambda j: (0, j)),
        ],
        out_specs=pl.BlockSpec((B * S, wo_blk), lambda j: (0, j)),
    )(o2, Wo)

    return out.reshape(B, S, D)


# device time: 87660 ns/iter; 1.7224x vs baseline; 1.4882x over previous
import jax
import jax.numpy as jnp
from jax import lax
from jax.experimental import pallas as pl
from jax.experimental.pallas import tpu as pltpu

B, S, H, Dh, Dr = 2, 512, 16, 128, 32
D = 2048
DC_SH = 128
HG = 4
F32 = jnp.float32
BF16 = jnp.bfloat16
SCALE = (Dh + Dr) ** -0.5


def _dot(a, b):
    return jnp.dot(a, b, preferred_element_type=F32)


def _comm_proj_body(x_ref, wdkv_ref, wuk_ref, wuv_ref, wkr_ref, wq_ref,
                    wqr_ref,
                    k_ref, v_ref, kr_ref, q_ref, qr_ref,
                    c_ref, wuk_bf, wuv_bf, peer_c, peer_wuk, peer_wuv,
                    send_sems, recv_sems):
    my_x = lax.axis_index("x")
    my_y = lax.axis_index("y")
    my_z = lax.axis_index("z")
    peer = (my_x, 1 - my_y, my_z)

    barrier_sem = pltpu.get_barrier_semaphore()
    pl.semaphore_signal(barrier_sem, inc=1, device_id=peer,
                        device_id_type=pl.DeviceIdType.MESH)
    pl.semaphore_wait(barrier_sem, 1)

    x_bf = x_ref[...].astype(BF16)
    wuk_bf[...] = wuk_ref[...].astype(BF16)
    wuv_bf[...] = wuv_ref[...].astype(BF16)
    c_ref[...] = _dot(x_bf, wdkv_ref[...].astype(BF16)).astype(BF16)

    rdmas = []
    for i, (src, dst) in enumerate(((c_ref, peer_c),
                                    (wuk_bf, peer_wuk),
                                    (wuv_bf, peer_wuv))):
        rdma = pltpu.make_async_remote_copy(
            src_ref=src, dst_ref=dst,
            send_sem=send_sems.at[i], recv_sem=recv_sems.at[i],
            device_id=peer, device_id_type=pl.DeviceIdType.MESH)
        rdma.start()
        rdmas.append(rdma)

    q_ref[...] = (_dot(x_bf, wq_ref[...].astype(BF16)) * SCALE).astype(BF16)
    qr_ref[...] = (_dot(x_bf, wqr_ref[...].astype(BF16)) * SCALE).astype(BF16)
    kr_ref[...] = _dot(x_bf, wkr_ref[...].astype(BF16)).astype(BF16)
    k_local = _dot(c_ref[...], wuk_bf[...])
    v_local = _dot(c_ref[...], wuv_bf[...])

    for rdma in rdmas:
        rdma.wait()

    k_ref[...] = (k_local + _dot(peer_c[...], peer_wuk[...])).astype(BF16)
    v_ref[...] = (v_local + _dot(peer_c[...], peer_wuv[...])).astype(BF16)


def _attn_body(q_ref, qr_ref, k_ref, v_ref, kr_ref, o_ref):
    kr = kr_ref[0]
    contract_last = (((1,), (1,)), ((), ()))
    for i in range(HG):
        q = q_ref[0][:, i * Dh:(i + 1) * Dh]
        qr = qr_ref[0][:, i * Dr:(i + 1) * Dr]
        k = k_ref[0][:, i * Dh:(i + 1) * Dh]
        v = v_ref[0][:, i * Dh:(i + 1) * Dh]
        scores = (lax.dot_general(q, k, contract_last,
                                  preferred_element_type=F32)
                  + lax.dot_general(qr, kr, contract_last,
                                    preferred_element_type=F32))
        p = jnp.exp(scores)
        recip = 1.0 / jnp.sum(p, axis=-1, keepdims=True)
        o = _dot(p.astype(BF16), v)
        o_ref[0, :, i * Dh:(i + 1) * Dh] = (o * recip).astype(BF16)


def _out_body(o_ref, wo_ref, out_ref):
    out_ref[...] = _dot(o_ref[...], wo_ref[...].astype(BF16))


def kernel(x, Wdkv, Wuk, Wuv, Wq, Wqr, Wkr, Wo):
    x2 = x.reshape(B * S, D)

    k, v, kr, q, qr = pl.pallas_call(
        _comm_proj_body,
        out_shape=(
            jax.ShapeDtypeStruct((B * S, H * Dh), BF16),
            jax.ShapeDtypeStruct((B * S, H * Dh), BF16),
            jax.ShapeDtypeStruct((B * S, Dr), BF16),
            jax.ShapeDtypeStruct((B * S, H * Dh), BF16),
            jax.ShapeDtypeStruct((B * S, H * Dr), BF16),
        ),
        in_specs=[pl.BlockSpec(memory_space=pltpu.VMEM)] * 7,
        out_specs=[pl.BlockSpec(memory_space=pltpu.VMEM)] * 5,
        scratch_shapes=[
            pltpu.VMEM((B * S, DC_SH), BF16),
            pltpu.VMEM((DC_SH, H * Dh), BF16),
            pltpu.VMEM((DC_SH, H * Dh), BF16),
            pltpu.VMEM((B * S, DC_SH), BF16),
            pltpu.VMEM((DC_SH, H * Dh), BF16),
            pltpu.VMEM((DC_SH, H * Dh), BF16),
            pltpu.SemaphoreType.DMA((3,)),
            pltpu.SemaphoreType.DMA((3,)),
        ],
        compiler_params=pltpu.CompilerParams(
            collective_id=0, vmem_limit_bytes=100 * 1024 * 1024),
    )(x2, Wdkv, Wuk, Wuv, Wkr, Wq, Wqr)

    k3 = k.reshape(B, S, H * Dh)
    v3 = v.reshape(B, S, H * Dh)
    kr3 = kr.reshape(B, S, Dr)
    q3 = q.reshape(B, S, H * Dh)
    qr3 = qr.reshape(B, S, H * Dr)

    n_hg = H // HG
    o = pl.pallas_call(
        _attn_body,
        grid=(B, n_hg),
        out_shape=jax.ShapeDtypeStruct((B, S, H * Dh), BF16),
        in_specs=[
            pl.BlockSpec((1, S, HG * Dh), lambda b, g: (b, 0, g)),
            pl.BlockSpec((1, S, HG * Dr), lambda b, g: (b, 0, g)),
            pl.BlockSpec((1, S, HG * Dh), lambda b, g: (b, 0, g)),
            pl.BlockSpec((1, S, HG * Dh), lambda b, g: (b, 0, g)),
            pl.BlockSpec((1, S, Dr), lambda b, g: (b, 0, 0)),
        ],
        out_specs=pl.BlockSpec((1, S, HG * Dh), lambda b, g: (b, 0, g)),
    )(q3, qr3, k3, v3, kr3)

    o2 = o.reshape(B * S, H * Dh)
    n_wo_blocks = 4
    wo_blk = D // n_wo_blocks
    out = pl.pallas_call(
        _out_body,
        grid=(n_wo_blocks,),
        out_shape=jax.ShapeDtypeStruct((B * S, D), F32),
        in_specs=[
            pl.BlockSpec((B * S, H * Dh), lambda j: (0, 0)),
            pl.BlockSpec((H * Dh, wo_blk), lambda j: (0, j)),
        ],
        out_specs=pl.BlockSpec((B * S, wo_blk), lambda j: (0, j)),
    )(o2, Wo)

    return out.reshape(B, S, D)


# device time: 73627 ns/iter; 2.0507x vs baseline; 1.1906x over previous
import jax
import jax.numpy as jnp
from jax import lax
from jax.experimental import pallas as pl
from jax.experimental.pallas import tpu as pltpu

B, S, H, Dh, Dr = 2, 512, 16, 128, 32
D = 2048
DC_SH = 128
HG = 8
NB = 4
WQB = D // NB
F32 = jnp.float32
BF16 = jnp.bfloat16
SCALE = (Dh + Dr) ** -0.5


def _dot(a, b):
    return jnp.dot(a, b, preferred_element_type=F32)


def _comm_proj_body(x_ref, wdkv_ref, wuk_ref, wuv_ref, wkr_ref, wq_ref,
                    wqr_ref,
                    k_ref, v_ref, kr_ref, q_ref, qr_ref,
                    xbf, c_ref, wuk_bf, wuv_bf, peer_c, peer_wuk, peer_wuv,
                    send_sems, recv_sems):
    g = pl.program_id(0)
    my_x = lax.axis_index("x")
    my_y = lax.axis_index("y")
    my_z = lax.axis_index("z")
    peer = (my_x, 1 - my_y, my_z)

    def exchange_rdmas():
        return [
            pltpu.make_async_remote_copy(
                src_ref=src, dst_ref=dst,
                send_sem=send_sems.at[i], recv_sem=recv_sems.at[i],
                device_id=peer, device_id_type=pl.DeviceIdType.MESH)
            for i, (src, dst) in enumerate(((c_ref, peer_c),
                                            (wuk_bf, peer_wuk),
                                            (wuv_bf, peer_wuv)))
        ]

    @pl.when(g == 0)
    def _():
        barrier_sem = pltpu.get_barrier_semaphore()
        pl.semaphore_signal(barrier_sem, inc=1, device_id=peer,
                            device_id_type=pl.DeviceIdType.MESH)
        pl.semaphore_wait(barrier_sem, 1)

        xbf[...] = x_ref[...].astype(BF16)
        wuk_bf[...] = wuk_ref[...].astype(BF16)
        wuv_bf[...] = wuv_ref[...].astype(BF16)
        c_ref[...] = _dot(xbf[...], wdkv_ref[...].astype(BF16)).astype(BF16)

        for rdma in exchange_rdmas():
            rdma.start()

        kr_ref[...] = _dot(xbf[...], wkr_ref[...].astype(BF16)).astype(BF16)
        qr_ref[...] = (_dot(xbf[...], wqr_ref[...].astype(BF16))
                       * SCALE).astype(BF16)
        k_ref[...] = _dot(c_ref[...], wuk_bf[...]).astype(BF16)
        v_ref[...] = _dot(c_ref[...], wuv_bf[...]).astype(BF16)

    q_ref[...] = (_dot(xbf[...], wq_ref[...].astype(BF16)) * SCALE
                  ).astype(BF16)

    @pl.when(g == NB - 1)
    def _():
        for rdma in exchange_rdmas():
            rdma.wait()
        k_ref[...] = (k_ref[...]
                      + _dot(peer_c[...], peer_wuk[...])).astype(BF16)
        v_ref[...] = (v_ref[...]
                      + _dot(peer_c[...], peer_wuv[...])).astype(BF16)


def _attn_body(q_ref, qr_ref, k_ref, v_ref, kr_ref, o_ref):
    kr = kr_ref[0]
    contract_last = (((1,), (1,)), ((), ()))
    for i in range(HG):
        q = q_ref[0][:, i * Dh:(i + 1) * Dh]
        qr = qr_ref[0][:, i * Dr:(i + 1) * Dr]
        k = k_ref[0][:, i * Dh:(i + 1) * Dh]
        v = v_ref[0][:, i * Dh:(i + 1) * Dh]
        scores = (lax.dot_general(q, k, contract_last,
                                  preferred_element_type=F32)
                  + lax.dot_general(qr, kr, contract_last,
                                    preferred_element_type=F32))
        p = jnp.exp(scores)
        recip = 1.0 / jnp.sum(p, axis=-1, keepdims=True)
        o = _dot(p.astype(BF16), v)
        o_ref[0, :, i * Dh:(i + 1) * Dh] = (o * recip).astype(BF16)


def _out_body(o_ref, wo_ref, out_ref):
    out_ref[...] = _dot(o_ref[...], wo_ref[...].astype(BF16))


def kernel(x, Wdkv, Wuk, Wuv, Wq, Wqr, Wkr, Wo):
    x2 = x.reshape(B * S, D)

    k, v, kr, q, qr = pl.pallas_call(
        _comm_proj_body,
        grid=(NB,),
        out_shape=(
            jax.ShapeDtypeStruct((B * S, H * Dh), BF16),
            jax.ShapeDtypeStruct((B * S, H * Dh), BF16),
            jax.ShapeDtypeStruct((B * S, Dr), BF16),
            jax.ShapeDtypeStruct((B * S, H * Dh), BF16),
            jax.ShapeDtypeStruct((B * S, H * Dr), BF16),
        ),
        in_specs=[
            pl.BlockSpec((B * S, D), lambda g: (0, 0)),
            pl.BlockSpec((D, DC_SH), lambda g: (0, 0)),
            pl.BlockSpec((DC_SH, H * Dh), lambda g: (0, 0)),
            pl.BlockSpec((DC_SH, H * Dh), lambda g: (0, 0)),
            pl.BlockSpec((D, Dr), lambda g: (0, 0)),
            pl.BlockSpec((D, WQB), lambda g: (0, g)),
            pl.BlockSpec((D, H * Dr), lambda g: (0, 0)),
        ],
        out_specs=[
            pl.BlockSpec((B * S, H * Dh), lambda g: (0, 0)),
            pl.BlockSpec((B * S, H * Dh), lambda g: (0, 0)),
            pl.BlockSpec((B * S, Dr), lambda g: (0, 0)),
            pl.BlockSpec((B * S, WQB), lambda g: (0, g)),
            pl.BlockSpec((B * S, H * Dr), lambda g: (0, 0)),
        ],
        scratch_shapes=[
            pltpu.VMEM((B * S, D), BF16),
            pltpu.VMEM((B * S, DC_SH), BF16),
            pltpu.VMEM((DC_SH, H * Dh), BF16),
            pltpu.VMEM((DC_SH, H * Dh), BF16),
            pltpu.VMEM((B * S, DC_SH), BF16),
            pltpu.VMEM((DC_SH, H * Dh), BF16),
            pltpu.VMEM((DC_SH, H * Dh), BF16),
            pltpu.SemaphoreType.DMA((3,)),
            pltpu.SemaphoreType.DMA((3,)),
        ],
        compiler_params=pltpu.CompilerParams(
            collective_id=0, vmem_limit_bytes=100 * 1024 * 1024),
    )(x2, Wdkv, Wuk, Wuv, Wkr, Wq, Wqr)

    k3 = k.reshape(B, S, H * Dh)
    v3 = v.reshape(B, S, H * Dh)
    kr3 = kr.reshape(B, S, Dr)
    q3 = q.reshape(B, S, H * Dh)
    qr3 = qr.reshape(B, S, H * Dr)

    n_hg = H // HG
    o = pl.pallas_call(
        _attn_body,
        grid=(B, n_hg),
        out_shape=jax.ShapeDtypeStruct((B, S, H * Dh), BF16),
        in_specs=[
            pl.BlockSpec((1, S, HG * Dh), lambda b, g: (b, 0, g)),
            pl.BlockSpec((1, S, HG * Dr), lambda b, g: (b, 0, g)),
            pl.BlockSpec((1, S, HG * Dh), lambda b, g: (b, 0, g)),
            pl.BlockSpec((1, S, HG * Dh), lambda b, g: (b, 0, g)),
            pl.BlockSpec((1, S, Dr), lambda b, g: (b, 0, 0)),
        ],
        out_specs=pl.BlockSpec((1, S, HG * Dh), lambda b, g: (b, 0, g)),
        compiler_params=pltpu.CompilerParams(
            vmem_limit_bytes=100 * 1024 * 1024),
    )(q3, qr3, k3, v3, kr3)

    o2 = o.reshape(B * S, H * Dh)
    n_wo_blocks = 4
    wo_blk = D // n_wo_blocks
    out = pl.pallas_call(
        _out_body,
        grid=(n_wo_blocks,),
        out_shape=jax.ShapeDtypeStruct((B * S, D), F32),
        in_specs=[
            pl.BlockSpec((B * S, H * Dh), lambda j: (0, 0)),
            pl.BlockSpec((H * Dh, wo_blk), lambda j: (0, j)),
        ],
        out_specs=pl.BlockSpec((B * S, wo_blk), lambda j: (0, j)),
        compiler_params=pltpu.CompilerParams(
            vmem_limit_bytes=100 * 1024 * 1024),
    )(o2, Wo)

    return out.reshape(B, S, D)
